# baseline (speedup 1.0000x reference)
import functools

import jax
import jax.numpy as jnp
from jax.experimental import pallas as pl
from jax.experimental.pallas import tpu as pltpu

NN = 1024
BS = 4 * NN
F0 = 64
H = 128
NC = 6


def _center(x):
    return x - jnp.mean(x, axis=1, keepdims=True)


def _cov(a, b):
    g = jax.lax.dot_general(a, b, (((1,), (1,)), ((), ())),
                            preferred_element_type=jnp.float32)
    return g / jnp.float32(F0 - 1)


def _dvec_kernel(thr_ref, x_ref, dcol_ref, drow_ref, m_ref):
    i = pl.program_id(0)
    xc = _center(x_ref[...])
    g = _cov(xc, xc)
    rows = jax.lax.broadcasted_iota(jnp.int32, (NN, NN), 0)
    cols = jax.lax.broadcasted_iota(jnp.int32, (NN, NN), 1)
    eyef = (rows == cols).astype(jnp.float32)
    ge = g * eyef
    dcol = jnp.sqrt(jnp.sum(ge, axis=1, keepdims=True))
    drow = jnp.sqrt(jnp.sum(ge, axis=0, keepdims=True))
    dcol_ref[...] = dcol
    drow_ref[...] = drow[None]
    c = g / dcol / drow
    noteye = (rows != cols).astype(jnp.bfloat16)
    m_ref[...] = ((jnp.abs(c) > thr_ref[0, i]).astype(jnp.bfloat16)
                  * noteye)[None]


def _masks_kernel(thr_ref, xa_ref, xb_ref, da_ref, db_ref, mult_ref, add_ref,
                  maskd_ref, m_ref):
    del maskd_ref
    s = pl.program_id(0)
    c = _cov(_center(xa_ref[...]), _center(xb_ref[...]))
    c = c / da_ref[...] / db_ref[0]
    th = thr_ref[0, jnp.where(s < 3, s + 5, 0)]
    msk = (jnp.abs(c) > th).astype(jnp.bfloat16) * mult_ref[0] + add_ref[0]
    m_ref[...] = msk[None]


def _maskt_kernel(src_ref, maskr_ref, m_ref):
    del maskr_ref
    m_ref[...] = jnp.transpose(src_ref[0], (1, 0))[None]


def _adj_kernel(ma_ref, c_ref, adj_ref):
    cnt = jax.lax.dot_general(ma_ref[0], c_ref[0], (((1,), (0,)), ((), ())),
                              preferred_element_type=jnp.float32)
    adj_ref[...] = (cnt > 0.0).astype(jnp.int8)


def _deg_kernel(adj_ref, nr_ref, nc_ref, scr_ref, scc_ref, degr_s, degc_s):
    i = pl.program_id(0)
    ab = adj_ref[...].astype(jnp.bfloat16)
    ones_r = jnp.ones((BS, 1), jnp.bfloat16)
    ones_l = jnp.ones((NN, 1), jnp.bfloat16)
    rowsum = jax.lax.dot_general(ab, ones_r, (((1,), (0,)), ((), ())),
                                 preferred_element_type=jnp.float32)
    colpart = jax.lax.dot_general(ab, ones_l, (((0,), (0,)), ((), ())),
                                  preferred_element_type=jnp.float32)
    degr_s[pl.ds(i * NN, NN), :] = rowsum

    @pl.when(i == 0)
    def _():
        degc_s[...] = colpart

    @pl.when(i > 0)
    def _():
        degc_s[...] = degc_s[...] + colpart

    @pl.when(i == 3)
    def _():
        degr = degr_s[...]
        degc = degc_s[...]
        n_r = jnp.where(degr > 0, jax.lax.rsqrt(jnp.maximum(degr, 1.0)), 0.0)
        n_c = jnp.where(degc > 0, jax.lax.rsqrt(jnp.maximum(degc, 1.0)), 0.0)
        nr_ref[...] = n_r
        nc_ref[...] = n_c
        scr_ref[...] = n_r / jnp.maximum(degr, 1.0)
        scc_ref[...] = n_c / jnp.maximum(degc, 1.0)


def _gin_block(adj_col, adj_row, x, x_d, n_c, n_r, scc_d, scr_d,
               w_ac, b_ac, w_ca, b_ca):
    u = (x * n_c).astype(jnp.bfloat16)
    v = (x * n_r).astype(jnp.bfloat16)
    agg_ac = jax.lax.dot_general(adj_col.astype(jnp.bfloat16), u,
                                 (((0,), (0,)), ((), ())),
                                 preferred_element_type=jnp.float32)
    agg_ca = jax.lax.dot_general(adj_row.astype(jnp.bfloat16), v,
                                 (((1,), (0,)), ((), ())),
                                 preferred_element_type=jnp.float32)
    agg_ac = agg_ac * scc_d
    agg_ca = agg_ca * scr_d
    z_ac = jax.nn.relu(
        jnp.dot(x_d + agg_ac, w_ac, preferred_element_type=jnp.float32) + b_ac)
    z_ca = jax.nn.relu(
        jnp.dot(x_d + agg_ca, w_ca, preferred_element_type=jnp.float32) + b_ca)
    return jnp.concatenate([z_ac, z_ca], axis=1)


def _l1_kernel(adj_col_ref, adj_row_ref, x_ref, xd_ref, nr_ref, nc_ref,
               scrd_ref, sccd_ref,
               wac_ref, bac_ref, wca_ref, bca_ref,
               h_ref, ss_ref, sq_ref):
    h_d = _gin_block(adj_col_ref[...], adj_row_ref[...], x_ref[...],
                     xd_ref[...], nc_ref[...], nr_ref[...],
                     sccd_ref[...], scrd_ref[...],
                     wac_ref[...], bac_ref[...], wca_ref[...], bca_ref[...])
    h_ref[...] = h_d
    ss_ref[...] = jnp.sum(h_d, axis=0, keepdims=True)[None]
    sq_ref[...] = jnp.sum(h_d * h_d, axis=0, keepdims=True)[None]


def _l2_kernel(adj_col_ref, adj_row_ref, h1_ref, h1d_ref, ss_ref, sq_ref,
               g_ref, b_ref, nr_ref, nc_ref, scrd_ref, sccd_ref,
               wac_ref, bac_ref, wca_ref, bca_ref,
               h_ref, ss2_ref, sq2_ref):
    mu = jnp.sum(ss_ref[...][:, 0, :], axis=0, keepdims=True) / BS
    msq = jnp.sum(sq_ref[...][:, 0, :], axis=0, keepdims=True) / BS
    var = msq - mu * mu
    scale = jax.lax.rsqrt(var + 1e-5) * g_ref[...]
    bias = b_ref[...]
    x = (h1_ref[...] - mu) * scale + bias
    x_d = (h1d_ref[...] - mu) * scale + bias
    h_d = _gin_block(adj_col_ref[...], adj_row_ref[...], x, x_d,
                     nc_ref[...], nr_ref[...], sccd_ref[...], scrd_ref[...],
                     wac_ref[...], bac_ref[...], wca_ref[...], bca_ref[...])
    h_ref[...] = h_d
    ss2_ref[...] = jnp.sum(h_d, axis=0, keepdims=True)[None]
    sq2_ref[...] = jnp.sum(h_d * h_d, axis=0, keepdims=True)[None]


def _out_kernel(c_ref, h2_ref, ss_ref, sq_ref, g_ref, b_ref, wout_ref,
                out_ref):
    mu = jnp.sum(ss_ref[...][:, 0, :], axis=0, keepdims=True) / BS
    msq = jnp.sum(sq_ref[...][:, 0, :], axis=0, keepdims=True) / BS
    var = msq - mu * mu
    scale = jax.lax.rsqrt(var + 1e-5) * g_ref[...]
    h = (h2_ref[...] - mu) * scale + b_ref[...]
    y = jnp.dot(h, wout_ref[...], preferred_element_type=jnp.float32)
    c0 = c_ref[0, 0]
    c1 = c_ref[0, 1]
    c2 = c_ref[0, 2]
    c3 = c_ref[0, 3]
    m = jnp.maximum(jnp.maximum(c0, c1), jnp.maximum(c2, c3))
    e0 = jnp.exp(c0 - m)
    e1 = jnp.exp(c1 - m)
    e2 = jnp.exp(c2 - m)
    e3 = jnp.exp(c3 - m)
    den = e0 + e1 + e2 + e3
    out_ref[...] = (y[0 * NN:1 * NN] * (e0 / den) +
                    y[1 * NN:2 * NN] * (e1 / den) +
                    y[2 * NN:3 * NN] * (e2 / den) +
                    y[3 * NN:4 * NN] * (e3 / den))


@functools.partial(jax.jit, static_argnames=())
def kernel(features, sparse, c_param, W_ac1, b_ac1, W_ca1, b_ca1,
           W_ac2, b_ac2, W_ca2, b_ca2, bn1_g, bn1_b, bn2_g, bn2_b, W_out):
    f32 = jnp.float32
    bf16 = jnp.bfloat16

    sig = jax.nn.sigmoid(sparse[:, 0])
    thr = jnp.stack([sig[1], sig[5], sig[8], sig[10],
                     jnp.float32(0.0), sig[4], sig[7], sig[9]])[None, :]

    eye_b = jnp.eye(NN, dtype=bf16)
    mult = jnp.stack([jnp.ones((NN, NN), bf16), jnp.zeros((NN, NN), bf16)])
    add = jnp.stack([jnp.zeros((NN, NN), bf16), eye_b])

    dcol, drow, masks0 = pl.pallas_call(
        _dvec_kernel,
        grid=(4,),
        in_specs=[
            pl.BlockSpec(memory_space=pltpu.SMEM),
            pl.BlockSpec((NN, F0), lambda i: (i, 0)),
        ],
        out_specs=[pl.BlockSpec((NN, 1), lambda i: (i, 0)),
                   pl.BlockSpec((1, 1, NN), lambda i: (i, 0, 0)),
                   pl.BlockSpec((1, NN, NN), lambda i: (i, 0, 0))],
        out_shape=[jax.ShapeDtypeStruct((BS, 1), f32),
                   jax.ShapeDtypeStruct((4, 1, NN), f32),
                   jax.ShapeDtypeStruct((11, NN, NN), bf16)],
    )(thr, features)

    ia = lambda s: jnp.where(s < 3, s, 3)
    masks_r = pl.pallas_call(
        _masks_kernel,
        grid=(4,),
        in_specs=[
            pl.BlockSpec(memory_space=pltpu.SMEM),
            pl.BlockSpec((NN, F0), lambda s: (ia(s), 0)),
            pl.BlockSpec((NN, F0), lambda s: (3, 0)),
            pl.BlockSpec((NN, 1), lambda s: (ia(s), 0)),
            pl.BlockSpec((1, 1, NN), lambda s: (3, 0, 0)),
            pl.BlockSpec((1, NN, NN),
                         lambda s: (jnp.where(s < 3, 0, 1), 0, 0)),
            pl.BlockSpec((1, NN, NN),
                         lambda s: (jnp.where(s < 3, 0, 1), 0, 0)),
            pl.BlockSpec(memory_space=pltpu.MemorySpace.HBM),
        ],
        out_specs=pl.BlockSpec((1, NN, NN),
                               lambda s: (jnp.where(s < 3, s + 4, 10), 0, 0)),
        out_shape=jax.ShapeDtypeStruct((11, NN, NN), bf16),
        input_output_aliases={7: 0},
    )(thr, features, features, dcol, drow, mult, add, masks0)

    masks = pl.pallas_call(
        _maskt_kernel,
        grid=(3,),
        in_specs=[
            pl.BlockSpec((1, NN, NN), lambda s: (s + 4, 0, 0)),
            pl.BlockSpec(memory_space=pltpu.MemorySpace.HBM),
        ],
        out_specs=pl.BlockSpec((1, NN, NN), lambda s: (s + 7, 0, 0)),
        out_shape=jax.ShapeDtypeStruct((11, NN, NN), bf16),
        input_output_aliases={1: 0},
    )(masks_r, masks_r)

    adj = pl.pallas_call(
        _adj_kernel,
        grid=(4, 4),
        in_specs=[
            pl.BlockSpec((1, NN, NN), lambda i, j: (i, 0, 0)),
            pl.BlockSpec((1, NN, NN),
                         lambda i, j: (jnp.where(i == j, 10,
                                                 jnp.where(i > j, 3 + i,
                                                           6 + j)), 0, 0)),
        ],
        out_specs=pl.BlockSpec((NN, NN), lambda i, j: (i, j)),
        out_shape=jax.ShapeDtypeStruct((BS, BS), jnp.int8),
    )(masks, masks)

    n_r, n_c, sc_r, sc_c = pl.pallas_call(
        _deg_kernel,
        grid=(4,),
        in_specs=[pl.BlockSpec((NN, BS), lambda i: (i, 0))],
        out_specs=[pl.BlockSpec((BS, 1), lambda i: (0, 0))] * 4,
        out_shape=[jax.ShapeDtypeStruct((BS, 1), f32)] * 4,
        scratch_shapes=[pltpu.VMEM((BS, 1), f32), pltpu.VMEM((BS, 1), f32)],
    )(adj)

    def layer_specs(feat):
        return [
            pl.BlockSpec((BS, NN), lambda d: (0, d)),
            pl.BlockSpec((NN, BS), lambda d: (d, 0)),
        ]

    b2 = lambda a: a[None, :]

    h1, ss1, sq1 = pl.pallas_call(
        _l1_kernel,
        grid=(4,),
        in_specs=layer_specs(F0) + [
            pl.BlockSpec((BS, F0), lambda d: (0, 0)),
            pl.BlockSpec((NN, F0), lambda d: (d, 0)),
            pl.BlockSpec((BS, 1), lambda d: (0, 0)),
            pl.BlockSpec((BS, 1), lambda d: (0, 0)),
            pl.BlockSpec((NN, 1), lambda d: (d, 0)),
            pl.BlockSpec((NN, 1), lambda d: (d, 0)),
            pl.BlockSpec((F0, H), lambda d: (0, 0)),
            pl.BlockSpec((1, H), lambda d: (0, 0)),
            pl.BlockSpec((F0, H), lambda d: (0, 0)),
            pl.BlockSpec((1, H), lambda d: (0, 0)),
        ],
        out_specs=[
            pl.BlockSpec((NN, 2 * H), lambda d: (d, 0)),
            pl.BlockSpec((1, 1, 2 * H), lambda d: (d, 0, 0)),
            pl.BlockSpec((1, 1, 2 * H), lambda d: (d, 0, 0)),
        ],
        out_shape=[
            jax.ShapeDtypeStruct((BS, 2 * H), f32),
            jax.ShapeDtypeStruct((4, 1, 2 * H), f32),
            jax.ShapeDtypeStruct((4, 1, 2 * H), f32),
        ],
    )(adj, adj, features, features, n_r, n_c, sc_r, sc_c,
      W_ac1, b2(b_ac1), W_ca1, b2(b_ca1))

    h2, ss2, sq2 = pl.pallas_call(
        _l2_kernel,
        grid=(4,),
        in_specs=layer_specs(2 * H) + [
            pl.BlockSpec((BS, 2 * H), lambda d: (0, 0)),
            pl.BlockSpec((NN, 2 * H), lambda d: (d, 0)),
            pl.BlockSpec((4, 1, 2 * H), lambda d: (0, 0, 0)),
            pl.BlockSpec((4, 1, 2 * H), lambda d: (0, 0, 0)),
            pl.BlockSpec((1, 2 * H), lambda d: (0, 0)),
            pl.BlockSpec((1, 2 * H), lambda d: (0, 0)),
            pl.BlockSpec((BS, 1), lambda d: (0, 0)),
            pl.BlockSpec((BS, 1), lambda d: (0, 0)),
            pl.BlockSpec((NN, 1), lambda d: (d, 0)),
            pl.BlockSpec((NN, 1), lambda d: (d, 0)),
            pl.BlockSpec((2 * H, H), lambda d: (0, 0)),
            pl.BlockSpec((1, H), lambda d: (0, 0)),
            pl.BlockSpec((2 * H, H), lambda d: (0, 0)),
            pl.BlockSpec((1, H), lambda d: (0, 0)),
        ],
        out_specs=[
            pl.BlockSpec((NN, 2 * H), lambda d: (d, 0)),
            pl.BlockSpec((1, 1, 2 * H), lambda d: (d, 0, 0)),
            pl.BlockSpec((1, 1, 2 * H), lambda d: (d, 0, 0)),
        ],
        out_shape=[
            jax.ShapeDtypeStruct((BS, 2 * H), f32),
            jax.ShapeDtypeStruct((4, 1, 2 * H), f32),
            jax.ShapeDtypeStruct((4, 1, 2 * H), f32),
        ],
    )(adj, adj, h1, h1, ss1, sq1, b2(bn1_g), b2(bn1_b),
      n_r, n_c, sc_r, sc_c,
      W_ac2, b2(b_ac2), W_ca2, b2(b_ca2))

    out = pl.pallas_call(
        _out_kernel,
        in_specs=[
            pl.BlockSpec(memory_space=pltpu.SMEM),
            pl.BlockSpec((BS, 2 * H), lambda: (0, 0)),
            pl.BlockSpec((4, 1, 2 * H), lambda: (0, 0, 0)),
            pl.BlockSpec((4, 1, 2 * H), lambda: (0, 0, 0)),
            pl.BlockSpec((1, 2 * H), lambda: (0, 0)),
            pl.BlockSpec((1, 2 * H), lambda: (0, 0)),
            pl.BlockSpec((2 * H, NC), lambda: (0, 0)),
        ],
        out_shape=jax.ShapeDtypeStruct((NN, NC), f32),
    )(c_param, h2, ss2, sq2, b2(bn2_g), b2(bn2_b), W_out)

    return out


# speedup vs baseline: 1.1307x; 1.1307x over previous
import functools

import jax
import jax.numpy as jnp
from jax.experimental import pallas as pl
from jax.experimental.pallas import tpu as pltpu

NN = 1024
BS = 4 * NN
F0 = 64
H = 128
NC = 6


def _center(x):
    return x - jnp.mean(x, axis=1, keepdims=True)


def _cov(a, b):
    g = jax.lax.dot_general(a, b, (((1,), (1,)), ((), ())),
                            preferred_element_type=jnp.float32)
    return g / jnp.float32(F0 - 1)


def _dvec_kernel(thr_ref, x_ref, dcol_ref, drow_ref, m_ref):
    i = pl.program_id(0)
    xc = _center(x_ref[...])
    g = _cov(xc, xc)
    rows = jax.lax.broadcasted_iota(jnp.int32, (NN, NN), 0)
    cols = jax.lax.broadcasted_iota(jnp.int32, (NN, NN), 1)
    eyef = (rows == cols).astype(jnp.float32)
    ge = g * eyef
    dcol = jnp.sqrt(jnp.sum(ge, axis=1, keepdims=True))
    drow = jnp.sqrt(jnp.sum(ge, axis=0, keepdims=True))
    dcol_ref[...] = dcol
    drow_ref[...] = drow[None]
    c = g / dcol / drow
    noteye = (rows != cols).astype(jnp.bfloat16)
    m_ref[...] = ((jnp.abs(c) > thr_ref[0, i]).astype(jnp.bfloat16)
                  * noteye)[None]


def _masks_kernel(thr_ref, xa_ref, xb_ref, da_ref, db_ref, mult_ref, add_ref,
                  maskd_ref, m_ref):
    del maskd_ref
    s = pl.program_id(0)
    c = _cov(_center(xa_ref[...]), _center(xb_ref[...]))
    c = c / da_ref[...] / db_ref[0]
    th = thr_ref[0, jnp.where(s < 3, s + 5, jnp.where(s < 6, s + 2, 0))]
    msk = (jnp.abs(c) > th).astype(jnp.bfloat16) * mult_ref[0] + add_ref[0]
    m_ref[...] = msk[None]


def _adj_kernel(ma_ref, c_ref, adj_ref, degr_ref, degc_ref):
    i = pl.program_id(0)
    j = pl.program_id(1)
    cnt = jax.lax.dot_general(ma_ref[0], c_ref[0], (((1,), (0,)), ((), ())),
                              preferred_element_type=jnp.float32)
    ind = cnt > 0.0
    adj_ref[...] = ind.astype(jnp.int8)
    ind_bf = ind.astype(jnp.bfloat16)
    ones_b = jnp.ones((NN, 1), jnp.bfloat16)
    rowpart = jax.lax.dot_general(ind_bf, ones_b, (((1,), (0,)), ((), ())),
                                  preferred_element_type=jnp.float32)
    colpart = jax.lax.dot_general(ind_bf, ones_b, (((0,), (0,)), ((), ())),
                                  preferred_element_type=jnp.float32)

    @pl.when(j == 0)
    def _():
        degr_ref[pl.ds(i * NN, NN), :] = rowpart

    @pl.when(j > 0)
    def _():
        degr_ref[pl.ds(i * NN, NN), :] += rowpart

    @pl.when(i == 0)
    def _():
        degc_ref[pl.ds(j * NN, NN), :] = colpart

    @pl.when(i > 0)
    def _():
        degc_ref[pl.ds(j * NN, NN), :] += colpart


def _norms(deg):
    n = jnp.where(deg > 0, jax.lax.rsqrt(jnp.maximum(deg, 1.0)), 0.0)
    return n, n / jnp.maximum(deg, 1.0)


def _gin_block(adj_col, adj_row, x, x_d, n_c, n_r, scc_d, scr_d,
               w_ac, b_ac, w_ca, b_ca):
    u = (x * n_c).astype(jnp.bfloat16)
    v = (x * n_r).astype(jnp.bfloat16)
    agg_ac = jax.lax.dot_general(adj_col.astype(jnp.bfloat16), u,
                                 (((0,), (0,)), ((), ())),
                                 preferred_element_type=jnp.float32)
    agg_ca = jax.lax.dot_general(adj_row.astype(jnp.bfloat16), v,
                                 (((1,), (0,)), ((), ())),
                                 preferred_element_type=jnp.float32)
    agg_ac = agg_ac * scc_d
    agg_ca = agg_ca * scr_d
    z_ac = jax.nn.relu(
        jnp.dot(x_d + agg_ac, w_ac, preferred_element_type=jnp.float32) + b_ac)
    z_ca = jax.nn.relu(
        jnp.dot(x_d + agg_ca, w_ca, preferred_element_type=jnp.float32) + b_ca)
    return jnp.concatenate([z_ac, z_ca], axis=1)


def _l1_kernel(adj_col_ref, adj_row_ref, x_ref, xd_ref, degr_ref, degc_ref,
               degrd_ref, degcd_ref,
               wac_ref, bac_ref, wca_ref, bca_ref,
               h_ref, ss_ref, sq_ref):
    n_r, _ = _norms(degr_ref[...])
    n_c, _ = _norms(degc_ref[...])
    _, sc_r_d = _norms(degrd_ref[...])
    _, sc_c_d = _norms(degcd_ref[...])
    h_d = _gin_block(adj_col_ref[...], adj_row_ref[...], x_ref[...],
                     xd_ref[...], n_c, n_r, sc_c_d, sc_r_d,
                     wac_ref[...], bac_ref[...], wca_ref[...], bca_ref[...])
    h_ref[...] = h_d
    ss_ref[...] = jnp.sum(h_d, axis=0, keepdims=True)[None]
    sq_ref[...] = jnp.sum(h_d * h_d, axis=0, keepdims=True)[None]


def _l2_kernel(adj_col_ref, adj_row_ref, h1_ref, h1d_ref, ss_ref, sq_ref,
               g_ref, b_ref, degr_ref, degc_ref, degrd_ref, degcd_ref,
               wac_ref, bac_ref, wca_ref, bca_ref,
               h_ref, ss2_ref, sq2_ref):
    mu = jnp.sum(ss_ref[...][:, 0, :], axis=0, keepdims=True) / BS
    msq = jnp.sum(sq_ref[...][:, 0, :], axis=0, keepdims=True) / BS
    var = msq - mu * mu
    scale = jax.lax.rsqrt(var + 1e-5) * g_ref[...]
    bias = b_ref[...]
    x = (h1_ref[...] - mu) * scale + bias
    x_d = (h1d_ref[...] - mu) * scale + bias
    n_r, _ = _norms(degr_ref[...])
    n_c, _ = _norms(degc_ref[...])
    _, sc_r_d = _norms(degrd_ref[...])
    _, sc_c_d = _norms(degcd_ref[...])
    h_d = _gin_block(adj_col_ref[...], adj_row_ref[...], x, x_d,
                     n_c, n_r, sc_c_d, sc_r_d,
                     wac_ref[...], bac_ref[...], wca_ref[...], bca_ref[...])
    h_ref[...] = h_d
    ss2_ref[...] = jnp.sum(h_d, axis=0, keepdims=True)[None]
    sq2_ref[...] = jnp.sum(h_d * h_d, axis=0, keepdims=True)[None]


def _out_kernel(c_ref, h2_ref, ss_ref, sq_ref, g_ref, b_ref, wout_ref,
                out_ref):
    mu = jnp.sum(ss_ref[...][:, 0, :], axis=0, keepdims=True) / BS
    msq = jnp.sum(sq_ref[...][:, 0, :], axis=0, keepdims=True) / BS
    var = msq - mu * mu
    scale = jax.lax.rsqrt(var + 1e-5) * g_ref[...]
    h = (h2_ref[...] - mu) * scale + b_ref[...]
    y = jnp.dot(h, wout_ref[...], preferred_element_type=jnp.float32)
    c0 = c_ref[0, 0]
    c1 = c_ref[0, 1]
    c2 = c_ref[0, 2]
    c3 = c_ref[0, 3]
    m = jnp.maximum(jnp.maximum(c0, c1), jnp.maximum(c2, c3))
    e0 = jnp.exp(c0 - m)
    e1 = jnp.exp(c1 - m)
    e2 = jnp.exp(c2 - m)
    e3 = jnp.exp(c3 - m)
    den = e0 + e1 + e2 + e3
    out_ref[...] = (y[0 * NN:1 * NN] * (e0 / den) +
                    y[1 * NN:2 * NN] * (e1 / den) +
                    y[2 * NN:3 * NN] * (e2 / den) +
                    y[3 * NN:4 * NN] * (e3 / den))


@functools.partial(jax.jit, static_argnames=())
def kernel(features, sparse, c_param, W_ac1, b_ac1, W_ca1, b_ca1,
           W_ac2, b_ac2, W_ca2, b_ca2, bn1_g, bn1_b, bn2_g, bn2_b, W_out):
    f32 = jnp.float32
    bf16 = jnp.bfloat16

    sig = jax.nn.sigmoid(sparse[:, 0])
    thr = jnp.stack([sig[1], sig[5], sig[8], sig[10],
                     jnp.float32(0.0), sig[4], sig[7], sig[9]])[None, :]

    eye_b = jnp.eye(NN, dtype=bf16)
    mult = jnp.stack([jnp.ones((NN, NN), bf16), jnp.zeros((NN, NN), bf16)])
    add = jnp.stack([jnp.zeros((NN, NN), bf16), eye_b])

    dcol, drow, masks0 = pl.pallas_call(
        _dvec_kernel,
        grid=(4,),
        in_specs=[
            pl.BlockSpec(memory_space=pltpu.SMEM),
            pl.BlockSpec((NN, F0), lambda i: (i, 0)),
        ],
        out_specs=[pl.BlockSpec((NN, 1), lambda i: (i, 0)),
                   pl.BlockSpec((1, 1, NN), lambda i: (i, 0, 0)),
                   pl.BlockSpec((1, NN, NN), lambda i: (i, 0, 0))],
        out_shape=[jax.ShapeDtypeStruct((BS, 1), f32),
                   jax.ShapeDtypeStruct((4, 1, NN), f32),
                   jax.ShapeDtypeStruct((11, NN, NN), bf16)],
    )(thr, features)

    ia = lambda s: jnp.where(s < 3, s, 3)
    ib = lambda s: jnp.where(s < 3, 3, jnp.minimum(s - 3, 3))
    masks = pl.pallas_call(
        _masks_kernel,
        grid=(7,),
        in_specs=[
            pl.BlockSpec(memory_space=pltpu.SMEM),
            pl.BlockSpec((NN, F0), lambda s: (ia(s), 0)),
            pl.BlockSpec((NN, F0), lambda s: (ib(s), 0)),
            pl.BlockSpec((NN, 1), lambda s: (ia(s), 0)),
            pl.BlockSpec((1, 1, NN), lambda s: (ib(s), 0, 0)),
            pl.BlockSpec((1, NN, NN),
                         lambda s: (jnp.where(s < 6, 0, 1), 0, 0)),
            pl.BlockSpec((1, NN, NN),
                         lambda s: (jnp.where(s < 6, 0, 1), 0, 0)),
            pl.BlockSpec(memory_space=pltpu.MemorySpace.HBM),
        ],
        out_specs=pl.BlockSpec((1, NN, NN), lambda s: (s + 4, 0, 0)),
        out_shape=jax.ShapeDtypeStruct((11, NN, NN), bf16),
        input_output_aliases={7: 0},
    )(thr, features, features, dcol, drow, mult, add, masks0)

    adj, degr, degc = pl.pallas_call(
        _adj_kernel,
        grid=(4, 4),
        in_specs=[
            pl.BlockSpec((1, NN, NN), lambda i, j: (i, 0, 0)),
            pl.BlockSpec((1, NN, NN),
                         lambda i, j: (jnp.where(i == j, 10,
                                                 jnp.where(i > j, 3 + i,
                                                           6 + j)), 0, 0)),
        ],
        out_specs=[
            pl.BlockSpec((NN, NN), lambda i, j: (i, j)),
            pl.BlockSpec((BS, 1), lambda i, j: (0, 0)),
            pl.BlockSpec((BS, 1), lambda i, j: (0, 0)),
        ],
        out_shape=[
            jax.ShapeDtypeStruct((BS, BS), jnp.int8),
            jax.ShapeDtypeStruct((BS, 1), f32),
            jax.ShapeDtypeStruct((BS, 1), f32),
        ],
    )(masks, masks)

    def layer_specs(feat):
        return [
            pl.BlockSpec((BS, NN), lambda d: (0, d)),
            pl.BlockSpec((NN, BS), lambda d: (d, 0)),
        ]

    b2 = lambda a: a[None, :]

    h1, ss1, sq1 = pl.pallas_call(
        _l1_kernel,
        grid=(4,),
        in_specs=layer_specs(F0) + [
            pl.BlockSpec((BS, F0), lambda d: (0, 0)),
            pl.BlockSpec((NN, F0), lambda d: (d, 0)),
            pl.BlockSpec((BS, 1), lambda d: (0, 0)),
            pl.BlockSpec((BS, 1), lambda d: (0, 0)),
            pl.BlockSpec((NN, 1), lambda d: (d, 0)),
            pl.BlockSpec((NN, 1), lambda d: (d, 0)),
            pl.BlockSpec((F0, H), lambda d: (0, 0)),
            pl.BlockSpec((1, H), lambda d: (0, 0)),
            pl.BlockSpec((F0, H), lambda d: (0, 0)),
            pl.BlockSpec((1, H), lambda d: (0, 0)),
        ],
        out_specs=[
            pl.BlockSpec((NN, 2 * H), lambda d: (d, 0)),
            pl.BlockSpec((1, 1, 2 * H), lambda d: (d, 0, 0)),
            pl.BlockSpec((1, 1, 2 * H), lambda d: (d, 0, 0)),
        ],
        out_shape=[
            jax.ShapeDtypeStruct((BS, 2 * H), f32),
            jax.ShapeDtypeStruct((4, 1, 2 * H), f32),
            jax.ShapeDtypeStruct((4, 1, 2 * H), f32),
        ],
    )(adj, adj, features, features, degr, degc, degr, degc,
      W_ac1, b2(b_ac1), W_ca1, b2(b_ca1))

    h2, ss2, sq2 = pl.pallas_call(
        _l2_kernel,
        grid=(4,),
        in_specs=layer_specs(2 * H) + [
            pl.BlockSpec((BS, 2 * H), lambda d: (0, 0)),
            pl.BlockSpec((NN, 2 * H), lambda d: (d, 0)),
            pl.BlockSpec((4, 1, 2 * H), lambda d: (0, 0, 0)),
            pl.BlockSpec((4, 1, 2 * H), lambda d: (0, 0, 0)),
            pl.BlockSpec((1, 2 * H), lambda d: (0, 0)),
            pl.BlockSpec((1, 2 * H), lambda d: (0, 0)),
            pl.BlockSpec((BS, 1), lambda d: (0, 0)),
            pl.BlockSpec((BS, 1), lambda d: (0, 0)),
            pl.BlockSpec((NN, 1), lambda d: (d, 0)),
            pl.BlockSpec((NN, 1), lambda d: (d, 0)),
            pl.BlockSpec((2 * H, H), lambda d: (0, 0)),
            pl.BlockSpec((1, H), lambda d: (0, 0)),
            pl.BlockSpec((2 * H, H), lambda d: (0, 0)),
            pl.BlockSpec((1, H), lambda d: (0, 0)),
        ],
        out_specs=[
            pl.BlockSpec((NN, 2 * H), lambda d: (d, 0)),
            pl.BlockSpec((1, 1, 2 * H), lambda d: (d, 0, 0)),
            pl.BlockSpec((1, 1, 2 * H), lambda d: (d, 0, 0)),
        ],
        out_shape=[
            jax.ShapeDtypeStruct((BS, 2 * H), f32),
            jax.ShapeDtypeStruct((4, 1, 2 * H), f32),
            jax.ShapeDtypeStruct((4, 1, 2 * H), f32),
        ],
    )(adj, adj, h1, h1, ss1, sq1, b2(bn1_g), b2(bn1_b),
      degr, degc, degr, degc,
      W_ac2, b2(b_ac2), W_ca2, b2(b_ca2))

    out = pl.pallas_call(
        _out_kernel,
        in_specs=[
            pl.BlockSpec(memory_space=pltpu.SMEM),
            pl.BlockSpec((BS, 2 * H), lambda: (0, 0)),
            pl.BlockSpec((4, 1, 2 * H), lambda: (0, 0, 0)),
            pl.BlockSpec((4, 1, 2 * H), lambda: (0, 0, 0)),
            pl.BlockSpec((1, 2 * H), lambda: (0, 0)),
            pl.BlockSpec((1, 2 * H), lambda: (0, 0)),
            pl.BlockSpec((2 * H, NC), lambda: (0, 0)),
        ],
        out_shape=jax.ShapeDtypeStruct((NN, NC), f32),
    )(c_param, h2, ss2, sq2, b2(bn2_g), b2(bn2_b), W_out)

    return out


# speedup vs baseline: 1.2297x; 1.0875x over previous
import functools

import jax
import jax.numpy as jnp
from jax.experimental import pallas as pl
from jax.experimental.pallas import tpu as pltpu

NN = 1024
BS = 4 * NN
F0 = 64
H = 128
NC = 6


def _center(x):
    return x - jnp.mean(x, axis=1, keepdims=True)


def _cov(a, b):
    g = jax.lax.dot_general(a, b, (((1,), (1,)), ((), ())),
                            preferred_element_type=jnp.float32)
    return g / jnp.float32(F0 - 1)


def _dvec_kernel(thr_ref, x_ref, dcol_ref, drow_ref, m_ref):
    i = pl.program_id(0)
    xc = _center(x_ref[...])
    g = _cov(xc, xc)
    rows = jax.lax.broadcasted_iota(jnp.int32, (NN, NN), 0)
    cols = jax.lax.broadcasted_iota(jnp.int32, (NN, NN), 1)
    eyef = (rows == cols).astype(jnp.float32)
    ge = g * eyef
    dcol = jnp.sqrt(jnp.sum(ge, axis=1, keepdims=True))
    drow = jnp.sqrt(jnp.sum(ge, axis=0, keepdims=True))
    dcol_ref[...] = dcol
    drow_ref[...] = drow[None]
    c = g / dcol / drow
    noteye = (rows != cols).astype(jnp.bfloat16)
    m_ref[...] = ((jnp.abs(c) > thr_ref[0, i]).astype(jnp.bfloat16)
                  * noteye)[None]


def _masks_kernel(thr_ref, xa_ref, xb_ref, da_ref, db_ref, maskd_ref, m_ref):
    del maskd_ref
    s = pl.program_id(0)
    c = _cov(_center(xa_ref[...]), _center(xb_ref[...]))
    c = c / da_ref[...] / db_ref[0]
    th = thr_ref[0, jnp.where(s < 3, s + 5, s + 2)]
    m_ref[...] = (jnp.abs(c) > th).astype(jnp.bfloat16)[None]


def _adj_kernel(ma_ref, c_ref, adj_ref, degr_ref, degc_ref):
    i = pl.program_id(0)
    j = pl.program_id(1)
    ma = ma_ref[0]
    cnt = jax.lax.dot_general(ma, c_ref[0], (((1,), (0,)), ((), ())),
                              preferred_element_type=jnp.float32)
    ind_bf = jnp.where(i == j, ma, (cnt > 0.0).astype(jnp.bfloat16))
    adj_ref[...] = ind_bf.astype(jnp.int8)
    ones_b = jnp.ones((NN, 1), jnp.bfloat16)
    rowpart = jax.lax.dot_general(ind_bf, ones_b, (((1,), (0,)), ((), ())),
                                  preferred_element_type=jnp.float32)
    colpart = jax.lax.dot_general(ind_bf, ones_b, (((0,), (0,)), ((), ())),
                                  preferred_element_type=jnp.float32)

    @pl.when(j == 0)
    def _():
        degr_ref[pl.ds(i * NN, NN), :] = rowpart

    @pl.when(j > 0)
    def _():
        degr_ref[pl.ds(i * NN, NN), :] += rowpart

    @pl.when(i == 0)
    def _():
        degc_ref[pl.ds(j * NN, NN), :] = colpart

    @pl.when(i > 0)
    def _():
        degc_ref[pl.ds(j * NN, NN), :] += colpart


def _norms(deg):
    n = jnp.where(deg > 0, jax.lax.rsqrt(jnp.maximum(deg, 1.0)), 0.0)
    return n, n / jnp.maximum(deg, 1.0)


def _gin_block(adj_col, adj_row, x, x_d, n_c, n_r, scc_d, scr_d,
               w_ac, b_ac, w_ca, b_ca):
    u = (x * n_c).astype(jnp.bfloat16)
    v = (x * n_r).astype(jnp.bfloat16)
    agg_ac = jax.lax.dot_general(adj_col.astype(jnp.bfloat16), u,
                                 (((0,), (0,)), ((), ())),
                                 preferred_element_type=jnp.float32)
    agg_ca = jax.lax.dot_general(adj_row.astype(jnp.bfloat16), v,
                                 (((1,), (0,)), ((), ())),
                                 preferred_element_type=jnp.float32)
    agg_ac = agg_ac * scc_d
    agg_ca = agg_ca * scr_d
    z_ac = jax.nn.relu(
        jnp.dot(x_d + agg_ac, w_ac, preferred_element_type=jnp.float32) + b_ac)
    z_ca = jax.nn.relu(
        jnp.dot(x_d + agg_ca, w_ca, preferred_element_type=jnp.float32) + b_ca)
    return jnp.concatenate([z_ac, z_ca], axis=1)


def _l1_kernel(adj_col_ref, adj_row_ref, x_ref, xd_ref, degr_ref, degc_ref,
               degrd_ref, degcd_ref,
               wac_ref, bac_ref, wca_ref, bca_ref,
               h_ref, ss_ref, sq_ref):
    n_r, _ = _norms(degr_ref[...])
    n_c, _ = _norms(degc_ref[...])
    _, sc_r_d = _norms(degrd_ref[...])
    _, sc_c_d = _norms(degcd_ref[...])
    h_d = _gin_block(adj_col_ref[...], adj_row_ref[...], x_ref[...],
                     xd_ref[...], n_c, n_r, sc_c_d, sc_r_d,
                     wac_ref[...], bac_ref[...], wca_ref[...], bca_ref[...])
    h_ref[...] = h_d
    ss_ref[...] = jnp.sum(h_d, axis=0, keepdims=True)[None]
    sq_ref[...] = jnp.sum(h_d * h_d, axis=0, keepdims=True)[None]


def _l2_kernel(adj_col_ref, adj_row_ref, h1_ref, h1d_ref, ss_ref, sq_ref,
               g_ref, b_ref, degr_ref, degc_ref, degrd_ref, degcd_ref,
               wac_ref, bac_ref, wca_ref, bca_ref,
               h_ref, ss2_ref, sq2_ref):
    mu = jnp.sum(ss_ref[...][:, 0, :], axis=0, keepdims=True) / BS
    msq = jnp.sum(sq_ref[...][:, 0, :], axis=0, keepdims=True) / BS
    var = msq - mu * mu
    scale = jax.lax.rsqrt(var + 1e-5) * g_ref[...]
    bias = b_ref[...]
    x = (h1_ref[...] - mu) * scale + bias
    x_d = (h1d_ref[...] - mu) * scale + bias
    n_r, _ = _norms(degr_ref[...])
    n_c, _ = _norms(degc_ref[...])
    _, sc_r_d = _norms(degrd_ref[...])
    _, sc_c_d = _norms(degcd_ref[...])
    h_d = _gin_block(adj_col_ref[...], adj_row_ref[...], x, x_d,
                     n_c, n_r, sc_c_d, sc_r_d,
                     wac_ref[...], bac_ref[...], wca_ref[...], bca_ref[...])
    h_ref[...] = h_d
    ss2_ref[...] = jnp.sum(h_d, axis=0, keepdims=True)[None]
    sq2_ref[...] = jnp.sum(h_d * h_d, axis=0, keepdims=True)[None]


def _out_kernel(c_ref, h2_ref, ss_ref, sq_ref, g_ref, b_ref, wout_ref,
                out_ref):
    mu = jnp.sum(ss_ref[...][:, 0, :], axis=0, keepdims=True) / BS
    msq = jnp.sum(sq_ref[...][:, 0, :], axis=0, keepdims=True) / BS
    var = msq - mu * mu
    scale = jax.lax.rsqrt(var + 1e-5) * g_ref[...]
    h = (h2_ref[...] - mu) * scale + b_ref[...]
    y = jnp.dot(h, wout_ref[...], preferred_element_type=jnp.float32)
    c0 = c_ref[0, 0]
    c1 = c_ref[0, 1]
    c2 = c_ref[0, 2]
    c3 = c_ref[0, 3]
    m = jnp.maximum(jnp.maximum(c0, c1), jnp.maximum(c2, c3))
    e0 = jnp.exp(c0 - m)
    e1 = jnp.exp(c1 - m)
    e2 = jnp.exp(c2 - m)
    e3 = jnp.exp(c3 - m)
    den = e0 + e1 + e2 + e3
    out_ref[...] = (y[0 * NN:1 * NN] * (e0 / den) +
                    y[1 * NN:2 * NN] * (e1 / den) +
                    y[2 * NN:3 * NN] * (e2 / den) +
                    y[3 * NN:4 * NN] * (e3 / den))


@functools.partial(jax.jit, static_argnames=())
def kernel(features, sparse, c_param, W_ac1, b_ac1, W_ca1, b_ca1,
           W_ac2, b_ac2, W_ca2, b_ca2, bn1_g, bn1_b, bn2_g, bn2_b, W_out):
    f32 = jnp.float32
    bf16 = jnp.bfloat16

    sig = jax.nn.sigmoid(sparse[:, 0])
    thr = jnp.stack([sig[1], sig[5], sig[8], sig[10],
                     jnp.float32(0.0), sig[4], sig[7], sig[9]])[None, :]

    dcol, drow, masks0 = pl.pallas_call(
        _dvec_kernel,
        grid=(4,),
        in_specs=[
            pl.BlockSpec(memory_space=pltpu.SMEM),
            pl.BlockSpec((NN, F0), lambda i: (i, 0)),
        ],
        out_specs=[pl.BlockSpec((NN, 1), lambda i: (i, 0)),
                   pl.BlockSpec((1, 1, NN), lambda i: (i, 0, 0)),
                   pl.BlockSpec((1, NN, NN), lambda i: (i, 0, 0))],
        out_shape=[jax.ShapeDtypeStruct((BS, 1), f32),
                   jax.ShapeDtypeStruct((4, 1, NN), f32),
                   jax.ShapeDtypeStruct((10, NN, NN), bf16)],
    )(thr, features)

    ia = lambda s: jnp.where(s < 3, s, 3)
    ib = lambda s: jnp.where(s < 3, 3, s - 3)
    masks = pl.pallas_call(
        _masks_kernel,
        grid=(6,),
        in_specs=[
            pl.BlockSpec(memory_space=pltpu.SMEM),
            pl.BlockSpec((NN, F0), lambda s: (ia(s), 0)),
            pl.BlockSpec((NN, F0), lambda s: (ib(s), 0)),
            pl.BlockSpec((NN, 1), lambda s: (ia(s), 0)),
            pl.BlockSpec((1, 1, NN), lambda s: (ib(s), 0, 0)),
            pl.BlockSpec(memory_space=pltpu.MemorySpace.HBM),
        ],
        out_specs=pl.BlockSpec((1, NN, NN), lambda s: (s + 4, 0, 0)),
        out_shape=jax.ShapeDtypeStruct((10, NN, NN), bf16),
        input_output_aliases={5: 0},
    )(thr, features, features, dcol, drow, masks0)

    adj, degr, degc = pl.pallas_call(
        _adj_kernel,
        grid=(4, 4),
        in_specs=[
            pl.BlockSpec((1, NN, NN), lambda i, j: (i, 0, 0)),
            pl.BlockSpec((1, NN, NN),
                         lambda i, j: (jnp.where(i == j, 4,
                                                 jnp.where(i > j, 3 + i,
                                                           6 + j)), 0, 0)),
        ],
        out_specs=[
            pl.BlockSpec((NN, NN), lambda i, j: (i, j)),
            pl.BlockSpec((BS, 1), lambda i, j: (0, 0)),
            pl.BlockSpec((BS, 1), lambda i, j: (0, 0)),
        ],
        out_shape=[
            jax.ShapeDtypeStruct((BS, BS), jnp.int8),
            jax.ShapeDtypeStruct((BS, 1), f32),
            jax.ShapeDtypeStruct((BS, 1), f32),
        ],
    )(masks, masks)

    def layer_specs(feat):
        return [
            pl.BlockSpec((BS, NN), lambda d: (0, d)),
            pl.BlockSpec((NN, BS), lambda d: (d, 0)),
        ]

    b2 = lambda a: a[None, :]

    h1, ss1, sq1 = pl.pallas_call(
        _l1_kernel,
        grid=(4,),
        in_specs=layer_specs(F0) + [
            pl.BlockSpec((BS, F0), lambda d: (0, 0)),
            pl.BlockSpec((NN, F0), lambda d: (d, 0)),
            pl.BlockSpec((BS, 1), lambda d: (0, 0)),
            pl.BlockSpec((BS, 1), lambda d: (0, 0)),
            pl.BlockSpec((NN, 1), lambda d: (d, 0)),
            pl.BlockSpec((NN, 1), lambda d: (d, 0)),
            pl.BlockSpec((F0, H), lambda d: (0, 0)),
            pl.BlockSpec((1, H), lambda d: (0, 0)),
            pl.BlockSpec((F0, H), lambda d: (0, 0)),
            pl.BlockSpec((1, H), lambda d: (0, 0)),
        ],
        out_specs=[
            pl.BlockSpec((NN, 2 * H), lambda d: (d, 0)),
            pl.BlockSpec((1, 1, 2 * H), lambda d: (d, 0, 0)),
            pl.BlockSpec((1, 1, 2 * H), lambda d: (d, 0, 0)),
        ],
        out_shape=[
            jax.ShapeDtypeStruct((BS, 2 * H), f32),
            jax.ShapeDtypeStruct((4, 1, 2 * H), f32),
            jax.ShapeDtypeStruct((4, 1, 2 * H), f32),
        ],
    )(adj, adj, features, features, degr, degc, degr, degc,
      W_ac1, b2(b_ac1), W_ca1, b2(b_ca1))

    h2, ss2, sq2 = pl.pallas_call(
        _l2_kernel,
        grid=(4,),
        in_specs=layer_specs(2 * H) + [
            pl.BlockSpec((BS, 2 * H), lambda d: (0, 0)),
            pl.BlockSpec((NN, 2 * H), lambda d: (d, 0)),
            pl.BlockSpec((4, 1, 2 * H), lambda d: (0, 0, 0)),
            pl.BlockSpec((4, 1, 2 * H), lambda d: (0, 0, 0)),
            pl.BlockSpec((1, 2 * H), lambda d: (0, 0)),
            pl.BlockSpec((1, 2 * H), lambda d: (0, 0)),
            pl.BlockSpec((BS, 1), lambda d: (0, 0)),
            pl.BlockSpec((BS, 1), lambda d: (0, 0)),
            pl.BlockSpec((NN, 1), lambda d: (d, 0)),
            pl.BlockSpec((NN, 1), lambda d: (d, 0)),
            pl.BlockSpec((2 * H, H), lambda d: (0, 0)),
            pl.BlockSpec((1, H), lambda d: (0, 0)),
            pl.BlockSpec((2 * H, H), lambda d: (0, 0)),
            pl.BlockSpec((1, H), lambda d: (0, 0)),
        ],
        out_specs=[
            pl.BlockSpec((NN, 2 * H), lambda d: (d, 0)),
            pl.BlockSpec((1, 1, 2 * H), lambda d: (d, 0, 0)),
            pl.BlockSpec((1, 1, 2 * H), lambda d: (d, 0, 0)),
        ],
        out_shape=[
            jax.ShapeDtypeStruct((BS, 2 * H), f32),
            jax.ShapeDtypeStruct((4, 1, 2 * H), f32),
            jax.ShapeDtypeStruct((4, 1, 2 * H), f32),
        ],
    )(adj, adj, h1, h1, ss1, sq1, b2(bn1_g), b2(bn1_b),
      degr, degc, degr, degc,
      W_ac2, b2(b_ac2), W_ca2, b2(b_ca2))

    out = pl.pallas_call(
        _out_kernel,
        in_specs=[
            pl.BlockSpec(memory_space=pltpu.SMEM),
            pl.BlockSpec((BS, 2 * H), lambda: (0, 0)),
            pl.BlockSpec((4, 1, 2 * H), lambda: (0, 0, 0)),
            pl.BlockSpec((4, 1, 2 * H), lambda: (0, 0, 0)),
            pl.BlockSpec((1, 2 * H), lambda: (0, 0)),
            pl.BlockSpec((1, 2 * H), lambda: (0, 0)),
            pl.BlockSpec((2 * H, NC), lambda: (0, 0)),
        ],
        out_shape=jax.ShapeDtypeStruct((NN, NC), f32),
    )(c_param, h2, ss2, sq2, b2(bn2_g), b2(bn2_b), W_out)

    return out


# speedup vs baseline: 1.2312x; 1.0012x over previous
import functools

import jax
import jax.numpy as jnp
from jax.experimental import pallas as pl
from jax.experimental.pallas import tpu as pltpu

NN = 1024
BS = 4 * NN
F0 = 64
H = 128
NC = 6


def _center(x):
    return x - jnp.mean(x, axis=1, keepdims=True)


def _cov(a, b):
    g = jax.lax.dot_general(a, b, (((1,), (1,)), ((), ())),
                            preferred_element_type=jnp.float32)
    return g / jnp.float32(F0 - 1)


def _dvec_kernel(thr_ref, x_ref, dcol_ref, drow_ref, m_ref):
    i = pl.program_id(0)
    xc = _center(x_ref[...])
    g = _cov(xc, xc)
    rows = jax.lax.broadcasted_iota(jnp.int32, (NN, NN), 0)
    cols = jax.lax.broadcasted_iota(jnp.int32, (NN, NN), 1)
    eyef = (rows == cols).astype(jnp.float32)
    ge = g * eyef
    dcol = jnp.sqrt(jnp.sum(ge, axis=1, keepdims=True))
    drow = jnp.sqrt(jnp.sum(ge, axis=0, keepdims=True))
    dcol_ref[...] = dcol
    drow_ref[...] = drow[None]
    c = g / dcol / drow
    noteye = (rows != cols).astype(jnp.bfloat16)
    m_ref[...] = ((jnp.abs(c) > thr_ref[0, i]).astype(jnp.bfloat16)
                  * noteye)[None]


def _masks_kernel(thr_ref, xa_ref, xb_ref, da_ref, db_ref, maskd_ref, m_ref):
    del maskd_ref
    s = pl.program_id(0)
    c = _cov(_center(xa_ref[...]), _center(xb_ref[...]))
    c = c / da_ref[...] / db_ref[0]
    th = thr_ref[0, jnp.where(s < 3, s + 5, s + 2)]
    m_ref[...] = (jnp.abs(c) > th).astype(jnp.bfloat16)[None]


def _adj_kernel(ma_ref, c_ref, adj_ref, degr_ref, degc_ref):
    i = pl.program_id(0)
    j = pl.program_id(1)
    ma = ma_ref[0]
    cnt = jax.lax.dot_general(ma, c_ref[0], (((1,), (0,)), ((), ())),
                              preferred_element_type=jnp.float32)
    ind_bf = jnp.where(i == j, ma, (cnt > 0.0).astype(jnp.bfloat16))
    adj_ref[...] = ind_bf.astype(jnp.int8)
    ones_b = jnp.ones((NN, 1), jnp.bfloat16)
    rowpart = jax.lax.dot_general(ind_bf, ones_b, (((1,), (0,)), ((), ())),
                                  preferred_element_type=jnp.float32)
    colpart = jax.lax.dot_general(ind_bf, ones_b, (((0,), (0,)), ((), ())),
                                  preferred_element_type=jnp.float32)

    @pl.when(j == 0)
    def _():
        degr_ref[pl.ds(i * NN, NN), :] = rowpart

    @pl.when(j > 0)
    def _():
        degr_ref[pl.ds(i * NN, NN), :] += rowpart

    @pl.when(i == 0)
    def _():
        degc_ref[pl.ds(j * NN, NN), :] = colpart

    @pl.when(i > 0)
    def _():
        degc_ref[pl.ds(j * NN, NN), :] += colpart


def _norms(deg):
    n = jnp.where(deg > 0, jax.lax.rsqrt(jnp.maximum(deg, 1.0)), 0.0)
    return n, n / jnp.maximum(deg, 1.0)


def _gin_block(adj_col, adj_row, x, x_d, n_c, n_r, scc_d, scr_d,
               w_ac, b_ac, w_ca, b_ca):
    u = (x * n_c).astype(jnp.bfloat16)
    v = (x * n_r).astype(jnp.bfloat16)
    agg_ac = jax.lax.dot_general(adj_col.astype(jnp.bfloat16), u,
                                 (((0,), (0,)), ((), ())),
                                 preferred_element_type=jnp.float32)
    agg_ca = jax.lax.dot_general(adj_row.astype(jnp.bfloat16), v,
                                 (((1,), (0,)), ((), ())),
                                 preferred_element_type=jnp.float32)
    agg_ac = agg_ac * scc_d
    agg_ca = agg_ca * scr_d
    z_ac = jax.nn.relu(
        jnp.dot(x_d + agg_ac, w_ac, preferred_element_type=jnp.float32) + b_ac)
    z_ca = jax.nn.relu(
        jnp.dot(x_d + agg_ca, w_ca, preferred_element_type=jnp.float32) + b_ca)
    return jnp.concatenate([z_ac, z_ca], axis=1)


def _l1_kernel(adj_col_ref, adj_row_ref, x_ref, xd_ref, degr_ref, degc_ref,
               degrd_ref, degcd_ref,
               wac_ref, bac_ref, wca_ref, bca_ref,
               h_ref, ss_ref, sq_ref):
    n_r, _ = _norms(degr_ref[...])
    n_c, _ = _norms(degc_ref[...])
    _, sc_r_d = _norms(degrd_ref[...])
    _, sc_c_d = _norms(degcd_ref[...])
    h_d = _gin_block(adj_col_ref[...], adj_row_ref[...], x_ref[...],
                     xd_ref[...], n_c, n_r, sc_c_d, sc_r_d,
                     wac_ref[...], bac_ref[...], wca_ref[...], bca_ref[...])
    h_ref[...] = h_d
    ss_ref[...] = jnp.sum(h_d, axis=0, keepdims=True)[None]
    sq_ref[...] = jnp.sum(h_d * h_d, axis=0, keepdims=True)[None]


def _l2_kernel(adj_col_ref, adj_row_ref, h1_ref, h1d_ref, ss_ref, sq_ref,
               g_ref, b_ref, degr_ref, degc_ref, degrd_ref, degcd_ref,
               wac_ref, bac_ref, wca_ref, bca_ref,
               h_ref, ss2_ref, sq2_ref):
    mu = jnp.sum(ss_ref[...][:, 0, :], axis=0, keepdims=True) / BS
    msq = jnp.sum(sq_ref[...][:, 0, :], axis=0, keepdims=True) / BS
    var = msq - mu * mu
    scale = jax.lax.rsqrt(var + 1e-5) * g_ref[...]
    bias = b_ref[...]
    x = (h1_ref[...] - mu) * scale + bias
    x_d = (h1d_ref[...] - mu) * scale + bias
    n_r, _ = _norms(degr_ref[...])
    n_c, _ = _norms(degc_ref[...])
    _, sc_r_d = _norms(degrd_ref[...])
    _, sc_c_d = _norms(degcd_ref[...])
    h_d = _gin_block(adj_col_ref[...], adj_row_ref[...], x, x_d,
                     n_c, n_r, sc_c_d, sc_r_d,
                     wac_ref[...], bac_ref[...], wca_ref[...], bca_ref[...])
    h_ref[...] = h_d
    ss2_ref[...] = jnp.sum(h_d, axis=0, keepdims=True)[None]
    sq2_ref[...] = jnp.sum(h_d * h_d, axis=0, keepdims=True)[None]


def _out_kernel(c_ref, h2_ref, ss_ref, sq_ref, g_ref, b_ref, wout_ref,
                out_ref):
    mu = jnp.sum(ss_ref[...][:, 0, :], axis=0, keepdims=True) / BS
    msq = jnp.sum(sq_ref[...][:, 0, :], axis=0, keepdims=True) / BS
    var = msq - mu * mu
    scale = jax.lax.rsqrt(var + 1e-5) * g_ref[...]
    h = (h2_ref[...] - mu) * scale + b_ref[...]
    y = jnp.dot(h, wout_ref[...], preferred_element_type=jnp.float32)
    c0 = c_ref[0, 0]
    c1 = c_ref[0, 1]
    c2 = c_ref[0, 2]
    c3 = c_ref[0, 3]
    m = jnp.maximum(jnp.maximum(c0, c1), jnp.maximum(c2, c3))
    e0 = jnp.exp(c0 - m)
    e1 = jnp.exp(c1 - m)
    e2 = jnp.exp(c2 - m)
    e3 = jnp.exp(c3 - m)
    den = e0 + e1 + e2 + e3
    out_ref[...] = (y[0 * NN:1 * NN] * (e0 / den) +
                    y[1 * NN:2 * NN] * (e1 / den) +
                    y[2 * NN:3 * NN] * (e2 / den) +
                    y[3 * NN:4 * NN] * (e3 / den))


@functools.partial(jax.jit, static_argnames=())
def kernel(features, sparse, c_param, W_ac1, b_ac1, W_ca1, b_ca1,
           W_ac2, b_ac2, W_ca2, b_ca2, bn1_g, bn1_b, bn2_g, bn2_b, W_out):
    f32 = jnp.float32
    bf16 = jnp.bfloat16

    sig = jax.nn.sigmoid(sparse[:, 0])
    thr = jnp.stack([sig[1], sig[5], sig[8], sig[10],
                     jnp.float32(0.0), sig[4], sig[7], sig[9]])[None, :]

    dcol, drow, masks0 = pl.pallas_call(
        _dvec_kernel,
        grid=(4,),
        in_specs=[
            pl.BlockSpec(memory_space=pltpu.SMEM),
            pl.BlockSpec((NN, F0), lambda i: (i, 0)),
        ],
        out_specs=[pl.BlockSpec((NN, 1), lambda i: (i, 0)),
                   pl.BlockSpec((1, 1, NN), lambda i: (i, 0, 0)),
                   pl.BlockSpec((1, NN, NN), lambda i: (i, 0, 0))],
        out_shape=[jax.ShapeDtypeStruct((BS, 1), f32),
                   jax.ShapeDtypeStruct((4, 1, NN), f32),
                   jax.ShapeDtypeStruct((10, NN, NN), bf16)],
    )(thr, features)

    ia = lambda s: jnp.where(s < 3, s, 3)
    ib = lambda s: jnp.where(s < 3, 3, s - 3)
    masks = pl.pallas_call(
        _masks_kernel,
        grid=(6,),
        in_specs=[
            pl.BlockSpec(memory_space=pltpu.SMEM),
            pl.BlockSpec((NN, F0), lambda s: (ia(s), 0)),
            pl.BlockSpec((NN, F0), lambda s: (ib(s), 0)),
            pl.BlockSpec((NN, 1), lambda s: (ia(s), 0)),
            pl.BlockSpec((1, 1, NN), lambda s: (ib(s), 0, 0)),
            pl.BlockSpec(memory_space=pltpu.MemorySpace.HBM),
        ],
        out_specs=pl.BlockSpec((1, NN, NN), lambda s: (s + 4, 0, 0)),
        out_shape=jax.ShapeDtypeStruct((10, NN, NN), bf16),
        input_output_aliases={5: 0},
    )(thr, features, features, dcol, drow, masks0)

    adj, degr, degc = pl.pallas_call(
        _adj_kernel,
        grid=(4, 4),
        in_specs=[
            pl.BlockSpec((1, NN, NN), lambda i, j: (i, 0, 0)),
            pl.BlockSpec((1, NN, NN),
                         lambda i, j: (jnp.where(
                             i > j, 3 + i,
                             jnp.where(i < j, 6 + j,
                                       jnp.where(i == 0, 7, 3 + i))), 0, 0)),
        ],
        out_specs=[
            pl.BlockSpec((NN, NN), lambda i, j: (i, j)),
            pl.BlockSpec((BS, 1), lambda i, j: (0, 0)),
            pl.BlockSpec((BS, 1), lambda i, j: (0, 0)),
        ],
        out_shape=[
            jax.ShapeDtypeStruct((BS, BS), jnp.int8),
            jax.ShapeDtypeStruct((BS, 1), f32),
            jax.ShapeDtypeStruct((BS, 1), f32),
        ],
    )(masks, masks)

    def layer_specs(feat):
        return [
            pl.BlockSpec((BS, NN), lambda d: (0, d)),
            pl.BlockSpec((NN, BS), lambda d: (d, 0)),
        ]

    b2 = lambda a: a[None, :]

    h1, ss1, sq1 = pl.pallas_call(
        _l1_kernel,
        grid=(4,),
        in_specs=layer_specs(F0) + [
            pl.BlockSpec((BS, F0), lambda d: (0, 0)),
            pl.BlockSpec((NN, F0), lambda d: (d, 0)),
            pl.BlockSpec((BS, 1), lambda d: (0, 0)),
            pl.BlockSpec((BS, 1), lambda d: (0, 0)),
            pl.BlockSpec((NN, 1), lambda d: (d, 0)),
            pl.BlockSpec((NN, 1), lambda d: (d, 0)),
            pl.BlockSpec((F0, H), lambda d: (0, 0)),
            pl.BlockSpec((1, H), lambda d: (0, 0)),
            pl.BlockSpec((F0, H), lambda d: (0, 0)),
            pl.BlockSpec((1, H), lambda d: (0, 0)),
        ],
        out_specs=[
            pl.BlockSpec((NN, 2 * H), lambda d: (d, 0)),
            pl.BlockSpec((1, 1, 2 * H), lambda d: (d, 0, 0)),
            pl.BlockSpec((1, 1, 2 * H), lambda d: (d, 0, 0)),
        ],
        out_shape=[
            jax.ShapeDtypeStruct((BS, 2 * H), f32),
            jax.ShapeDtypeStruct((4, 1, 2 * H), f32),
            jax.ShapeDtypeStruct((4, 1, 2 * H), f32),
        ],
    )(adj, adj, features, features, degr, degc, degr, degc,
      W_ac1, b2(b_ac1), W_ca1, b2(b_ca1))

    h2, ss2, sq2 = pl.pallas_call(
        _l2_kernel,
        grid=(4,),
        in_specs=layer_specs(2 * H) + [
            pl.BlockSpec((BS, 2 * H), lambda d: (0, 0)),
            pl.BlockSpec((NN, 2 * H), lambda d: (d, 0)),
            pl.BlockSpec((4, 1, 2 * H), lambda d: (0, 0, 0)),
            pl.BlockSpec((4, 1, 2 * H), lambda d: (0, 0, 0)),
            pl.BlockSpec((1, 2 * H), lambda d: (0, 0)),
            pl.BlockSpec((1, 2 * H), lambda d: (0, 0)),
            pl.BlockSpec((BS, 1), lambda d: (0, 0)),
            pl.BlockSpec((BS, 1), lambda d: (0, 0)),
            pl.BlockSpec((NN, 1), lambda d: (d, 0)),
            pl.BlockSpec((NN, 1), lambda d: (d, 0)),
            pl.BlockSpec((2 * H, H), lambda d: (0, 0)),
            pl.BlockSpec((1, H), lambda d: (0, 0)),
            pl.BlockSpec((2 * H, H), lambda d: (0, 0)),
            pl.BlockSpec((1, H), lambda d: (0, 0)),
        ],
        out_specs=[
            pl.BlockSpec((NN, 2 * H), lambda d: (d, 0)),
            pl.BlockSpec((1, 1, 2 * H), lambda d: (d, 0, 0)),
            pl.BlockSpec((1, 1, 2 * H), lambda d: (d, 0, 0)),
        ],
        out_shape=[
            jax.ShapeDtypeStruct((BS, 2 * H), f32),
            jax.ShapeDtypeStruct((4, 1, 2 * H), f32),
            jax.ShapeDtypeStruct((4, 1, 2 * H), f32),
        ],
    )(adj, adj, h1, h1, ss1, sq1, b2(bn1_g), b2(bn1_b),
      degr, degc, degr, degc,
      W_ac2, b2(b_ac2), W_ca2, b2(b_ca2))

    out = pl.pallas_call(
        _out_kernel,
        in_specs=[
            pl.BlockSpec(memory_space=pltpu.SMEM),
            pl.BlockSpec((BS, 2 * H), lambda: (0, 0)),
            pl.BlockSpec((4, 1, 2 * H), lambda: (0, 0, 0)),
            pl.BlockSpec((4, 1, 2 * H), lambda: (0, 0, 0)),
            pl.BlockSpec((1, 2 * H), lambda: (0, 0)),
            pl.BlockSpec((1, 2 * H), lambda: (0, 0)),
            pl.BlockSpec((2 * H, NC), lambda: (0, 0)),
        ],
        out_shape=jax.ShapeDtypeStruct((NN, NC), f32),
    )(c_param, h2, ss2, sq2, b2(bn2_g), b2(bn2_b), W_out)

    return out
